# baseline (device time: 11633 ns/iter reference)
import jax
import jax.numpy as jnp
from jax import lax
from jax.experimental import pallas as pl
from jax.experimental.pallas import tpu as pltpu

N_DEV = 8
B, SQ, HQ, DH = 2, 128, 4, 64
DM = 512
DQ = HQ * DH
BLK = 64


def kernel(x, Wq, K_ext, V_ext, Wo):
    def body(q_ref, kv_ref, wo_ref, out_ref, ctx_ref, send_sems, recv_sems):
        my = lax.axis_index("i")

        barrier_sem = pltpu.get_barrier_semaphore()

        @pl.when(jnp.logical_and(my >= 1, my <= 4))
        def _():
            pl.semaphore_signal(barrier_sem, inc=1, device_id=(0,),
                                device_id_type=pl.DeviceIdType.MESH)

        @pl.when(my >= 5)
        def _():
            pl.semaphore_signal(barrier_sem, inc=1, device_id=(4,),
                                device_id_type=pl.DeviceIdType.MESH)

        def start_sends(b, targets):
            descs = []
            for i, j in enumerate(targets):
                rdma = pltpu.make_async_remote_copy(
                    src_ref=ctx_ref.at[b],
                    dst_ref=ctx_ref.at[b],
                    send_sem=send_sems.at[b, i],
                    recv_sem=recv_sems.at[b],
                    device_id=(j,),
                    device_id_type=pl.DeviceIdType.MESH,
                )
                rdma.start()
                descs.append(rdma)
            return descs

        def wait_recv(b):
            rdma = pltpu.make_async_remote_copy(
                src_ref=ctx_ref.at[b],
                dst_ref=ctx_ref.at[b],
                send_sem=send_sems.at[b, 0],
                recv_sem=recv_sems.at[b],
                device_id=(0,),
                device_id_type=pl.DeviceIdType.MESH,
            )
            rdma.wait_recv()

        def out_rows(b):
            out_ref[b, :, :] = jnp.dot(ctx_ref[b], wo_ref[...],
                                       preferred_element_type=jnp.float32)

        @pl.when(my == 0)
        def _():
            row = lax.broadcasted_iota(jnp.int32, (SQ, SQ), 0) // BLK
            col = lax.broadcasted_iota(jnp.int32, (SQ, SQ), 1) // BLK
            mask = col <= row
            sends = []
            for b in range(B):
                rows = slice(b * SQ, (b + 1) * SQ)
                for h in range(HQ):
                    cols = slice(h * DH, (h + 1) * DH)
                    base = (b * HQ + h) * DH
                    qh = q_ref[rows, cols]
                    kh = kv_ref[base:base + DH, :]
                    vh = kv_ref[B * DQ + base:B * DQ + base + DH, :]
                    s = jnp.dot(qh, kh,
                                preferred_element_type=jnp.float32) * 0.125
                    s = jnp.where(mask, s, -1e9)
                    m = jnp.max(s, axis=-1, keepdims=True)
                    w = jnp.exp(s - m)
                    w = (w / jnp.sum(w, axis=-1, keepdims=True)
                         ).astype(jnp.bfloat16)
                    ctx = lax.dot_general(
                        w, vh, (((1,), (1,)), ((), ())),
                        preferred_element_type=jnp.float32)
                    ctx_ref[b, :, cols] = ctx.astype(jnp.bfloat16)
                if b == 0:
                    pl.semaphore_wait(barrier_sem, 4)
                sends += start_sends(b, [1, 2, 3, 4])
                out_rows(b)
            for rdma in sends:
                rdma.wait_send()

        @pl.when(my == 4)
        def _():
            sends = []
            for b in range(B):
                wait_recv(b)
                if b == 0:
                    pl.semaphore_wait(barrier_sem, 3)
                sends += start_sends(b, [5, 6, 7])
                out_rows(b)
            for rdma in sends:
                rdma.wait_send()

        @pl.when(jnp.logical_and(my != 0, my != 4))
        def _():
            for b in range(B):
                wait_recv(b)
                out_rows(b)

    bf16 = jnp.bfloat16
    Q = jnp.dot(x.reshape(B * SQ, DM).astype(bf16), Wq.astype(bf16),
                preferred_element_type=bf16)

    K2 = jnp.transpose(K_ext, (0, 2, 3, 1)).reshape(B * DQ, SQ)
    V2 = jnp.transpose(V_ext, (0, 2, 3, 1)).reshape(B * DQ, SQ)
    KV = jnp.concatenate([K2, V2], axis=0).astype(bf16)

    vmem = pl.BlockSpec(memory_space=pltpu.MemorySpace.VMEM)
    return pl.pallas_call(
        body,
        out_shape=jax.ShapeDtypeStruct((B, SQ, DM), jnp.float32),
        in_specs=[vmem, vmem, vmem],
        out_specs=vmem,
        scratch_shapes=[
            pltpu.VMEM((B, SQ, DQ), jnp.bfloat16),
            pltpu.SemaphoreType.DMA((B, 4)),
            pltpu.SemaphoreType.DMA((B,)),
        ],
        compiler_params=pltpu.CompilerParams(collective_id=0),
    )(Q, KV, Wo.astype(bf16))


# device time: 10608 ns/iter; 1.0966x vs baseline; 1.0966x over previous
import jax
import jax.numpy as jnp
from jax import lax
from jax.experimental import pallas as pl
from jax.experimental.pallas import tpu as pltpu

N_DEV = 8
B, SQ, HQ, DH = 2, 128, 4, 64
DM = 512
DQ = HQ * DH
BLK = 64


def kernel(x, Wq, K_ext, V_ext, Wo):
    def body(q_ref, kv_ref, wo_ref, out_ref, ctx_ref, send_sems, recv_sems):
        my = lax.axis_index("i")

        barrier_sem = pltpu.get_barrier_semaphore()

        @pl.when(jnp.logical_and(my >= 1, my <= 4))
        def _():
            pl.semaphore_signal(barrier_sem, inc=1, device_id=(0,),
                                device_id_type=pl.DeviceIdType.MESH)

        @pl.when(my >= 5)
        def _():
            pl.semaphore_signal(barrier_sem, inc=1, device_id=(4,),
                                device_id_type=pl.DeviceIdType.MESH)

        def start_sends(b, targets):
            descs = []
            for i, j in enumerate(targets):
                rdma = pltpu.make_async_remote_copy(
                    src_ref=ctx_ref.at[b],
                    dst_ref=ctx_ref.at[b],
                    send_sem=send_sems.at[b, i],
                    recv_sem=recv_sems.at[b],
                    device_id=(j,),
                    device_id_type=pl.DeviceIdType.MESH,
                )
                rdma.start()
                descs.append(rdma)
            return descs

        def wait_recv(b):
            rdma = pltpu.make_async_remote_copy(
                src_ref=ctx_ref.at[b],
                dst_ref=ctx_ref.at[b],
                send_sem=send_sems.at[b, 0],
                recv_sem=recv_sems.at[b],
                device_id=(0,),
                device_id_type=pl.DeviceIdType.MESH,
            )
            rdma.wait_recv()

        def out_rows(b):
            ctxb = ctx_ref[b].astype(jnp.float32)
            out_ref[b, :, :] = jnp.dot(ctxb, wo_ref[...],
                                       preferred_element_type=jnp.float32)

        @pl.when(my == 0)
        def _():
            row = lax.broadcasted_iota(jnp.int32, (SQ, SQ), 0) // BLK
            col = lax.broadcasted_iota(jnp.int32, (SQ, SQ), 1) // BLK
            mask = col <= row
            sends = []
            for b in range(B):
                rows = slice(b * SQ, (b + 1) * SQ)
                for h in range(HQ):
                    cols = slice(h * DH, (h + 1) * DH)
                    base = (b * HQ + h) * DH
                    qh = q_ref[rows, cols]
                    kh = kv_ref[base:base + DH, :]
                    vh = kv_ref[B * DQ + base:B * DQ + base + DH, :]
                    s = jnp.dot(qh, kh,
                                preferred_element_type=jnp.float32) * 0.125
                    s = jnp.where(mask, s, -1e9)
                    m = jnp.max(s, axis=-1, keepdims=True)
                    w = jnp.exp(s - m)
                    w = w / jnp.sum(w, axis=-1, keepdims=True)
                    ctx = lax.dot_general(
                        w, vh, (((1,), (1,)), ((), ())),
                        preferred_element_type=jnp.float32)
                    ctx_ref[b, :, cols] = ctx.astype(jnp.bfloat16)
                if b == 0:
                    pl.semaphore_wait(barrier_sem, 4)
                sends += start_sends(b, [1, 2, 3, 4])
                out_rows(b)
            for rdma in sends:
                rdma.wait_send()

        @pl.when(my == 4)
        def _():
            sends = []
            for b in range(B):
                wait_recv(b)
                if b == 0:
                    pl.semaphore_wait(barrier_sem, 3)
                sends += start_sends(b, [5, 6, 7])
                out_rows(b)
            for rdma in sends:
                rdma.wait_send()

        @pl.when(jnp.logical_and(my != 0, my != 4))
        def _():
            for b in range(B):
                wait_recv(b)
                out_rows(b)

    Q = jnp.dot(x.reshape(B * SQ, DM), Wq)

    K2 = jnp.transpose(K_ext, (0, 2, 3, 1)).reshape(B * DQ, SQ)
    V2 = jnp.transpose(V_ext, (0, 2, 3, 1)).reshape(B * DQ, SQ)
    KV = jnp.concatenate([K2, V2], axis=0)

    vmem = pl.BlockSpec(memory_space=pltpu.MemorySpace.VMEM)
    return pl.pallas_call(
        body,
        out_shape=jax.ShapeDtypeStruct((B, SQ, DM), jnp.float32),
        in_specs=[vmem, vmem, vmem],
        out_specs=vmem,
        scratch_shapes=[
            pltpu.VMEM((B, SQ, DQ), jnp.bfloat16),
            pltpu.SemaphoreType.DMA((B, 4)),
            pltpu.SemaphoreType.DMA((B,)),
        ],
        compiler_params=pltpu.CompilerParams(collective_id=0),
    )(Q, KV, Wo)
